# Initial kernel scaffold; baseline (speedup 1.0000x reference)
import jax
import jax.numpy as jnp
from jax.experimental import pallas as pl


def kernel(x, edge_attr, embed_x_W, embed_edge_W):
    raise NotImplementedError("write your pallas kernel here")


# speedup vs baseline: 5.5924x; 5.5924x over previous
import jax
import jax.numpy as jnp
from jax import lax
from jax.experimental import pallas as pl
from jax.experimental.pallas import tpu as pltpu
from jax.experimental.pallas import tpu_sc as plsc

NC, NS = 2, 16
NW = NC * NS

NX, DX = 100000, 32
NE, DE = 3200000, 16
VE = 1000

CX = 1024
NSUB = CX // 128
NCHX = (NX + CX - 1) // CX
XLAST = NX - CX

BE = NE // NW
CE = 2000
NCHE = BE // CE
NG = CE // 16


def _body(x_hbm, e_hbm, wx_hbm, we_hbm, outx_hbm, oute_hbm,
          xidx_v, xrows_v, etab_v, eidx_v, erows_v, sem):
    wid = lax.axis_index("s") * NC + lax.axis_index("c")

    for i in range(4):
        c = wid + NW * i

        @pl.when(c < NCHX)
        def _():
            base = pl.multiple_of(jnp.minimum(c * CX, XLAST), 8)
            for j in range(NSUB):
                pltpu.sync_copy(x_hbm.at[pl.ds(base + j * 128, 128)],
                                xidx_v.at[j])
            cps = [pltpu.async_copy(wx_hbm.at[xidx_v.at[j]],
                                    xrows_v.at[pl.ds(j * 128, 128)], sem)
                   for j in range(NSUB)]
            for cp in cps:
                cp.wait()
            pltpu.sync_copy(xrows_v, outx_hbm.at[pl.ds(base, CX)])

    pltpu.sync_copy(we_hbm, etab_v)
    ebase = wid * BE
    lane = lax.iota(jnp.int32, 16)

    def echunk(i, carry):
        b = pl.multiple_of(ebase + i * CE, 8)
        pltpu.sync_copy(e_hbm.at[pl.ds(b, CE)], eidx_v)

        def egroup(g, carry2):
            rows16 = eidx_v[pl.ds(g * 16, 16)]
            dst_rows = g * 16 + lane
            for j in range(DE):
                col = jnp.full((16,), j, jnp.int32)
                vals = plsc.load_gather(etab_v, [rows16, col])
                plsc.store_scatter(erows_v, [dst_rows, col], vals)
            return carry2

        lax.fori_loop(0, NG, egroup, 0)
        pltpu.sync_copy(erows_v, oute_hbm.at[pl.ds(b, CE)])
        return carry

    lax.fori_loop(0, NCHE, echunk, 0)


def kernel(x, edge_attr, embed_x_W, embed_edge_W):
    mesh = plsc.VectorSubcoreMesh(core_axis_name="c", subcore_axis_name="s")
    f = pl.kernel(
        _body,
        out_type=[jax.ShapeDtypeStruct((NX, DX), jnp.float32),
                  jax.ShapeDtypeStruct((NE, DE), jnp.float32)],
        mesh=mesh,
        compiler_params=pltpu.CompilerParams(
            use_tc_tiling_on_sc=False, needs_layout_passes=False),
        scratch_types=[
            pltpu.VMEM((NSUB, 128), jnp.int32),
            pltpu.VMEM((CX, DX), jnp.float32),
            pltpu.VMEM((VE, DE), jnp.float32),
            pltpu.VMEM((CE,), jnp.int32),
            pltpu.VMEM((CE, DE), jnp.float32),
            pltpu.SemaphoreType.DMA,
        ],
    )
    out_x, out_e = f(x, edge_attr, embed_x_W, embed_edge_W)
    return (out_x, out_e)


# speedup vs baseline: 27.7211x; 4.9569x over previous
import jax
import jax.numpy as jnp
from jax import lax
from jax.experimental import pallas as pl
from jax.experimental.pallas import tpu as pltpu
from jax.experimental.pallas import tpu_sc as plsc

NC, NS = 2, 16
NW = NC * NS

NX, DX = 100000, 32
NE, DE = 3200000, 16
VE = 1000

CX = 1024
NSUB = CX // 128
NCHX = (NX + CX - 1) // CX
XLAST = NX - CX

NBLK = NE // 128
WBLK = 800
CE = 2048
NCHE = WBLK // 16
NG = CE // 16
EB_T = 16 * 8 * 128


def _body(x_hbm, e_hbm, wx_hbm, we_hbm, outx_hbm, oute_hbm,
          xidx_v, xrows_v, etab_v, eidx_v, erows_v, sem, si0, si1, sw0, sw1):
    wid = lax.axis_index("s") * NC + lax.axis_index("c")

    for i in range(4):
        c = wid + NW * i

        @pl.when(c < NCHX)
        def _():
            base = pl.multiple_of(jnp.minimum(c * CX, XLAST), 8)
            for j in range(NSUB):
                pltpu.sync_copy(x_hbm.at[pl.ds(base + j * 128, 128)],
                                xidx_v.at[j])
            cps = [pltpu.async_copy(wx_hbm.at[xidx_v.at[j]],
                                    xrows_v.at[pl.ds(j * 128, 128)], sem)
                   for j in range(NSUB)]
            for cp in cps:
                cp.wait()
            pltpu.sync_copy(xrows_v, outx_hbm.at[pl.ds(base, CX)])

    pltpu.sync_copy(we_hbm, etab_v)
    wblk0 = jnp.minimum(wid * WBLK, NBLK - WBLK)
    si = (si0, si1)
    sw = (sw0, sw1)

    def idx_start(i, bb):
        b = pl.multiple_of((wblk0 + i * 16) * 128, 8)
        pltpu.async_copy(e_hbm.at[pl.ds(b, CE)], eidx_v.at[bb], si[bb])

    def idx_wait(bb):
        pltpu.make_async_copy(e_hbm.at[pl.ds(0, CE)], eidx_v.at[bb],
                              si[bb]).wait()

    def write_start(i, bb):
        blk = wblk0 + i * 16
        for t in range(2):
            pltpu.async_copy(
                erows_v.at[bb, pl.ds(t * EB_T, EB_T)],
                oute_hbm.at[pl.ds((t * NBLK + blk) * 1024, EB_T)], sw[bb])

    def write_wait(bb):
        pltpu.make_async_copy(oute_hbm.at[pl.ds(0, 2 * EB_T)],
                              erows_v.at[bb], sw[bb]).wait()

    def compute(bb):
        @plsc.parallel_loop(0, NG, unroll=4)
        def _(g):
            rows16 = eidx_v[bb, pl.ds(g * 16, 16)]
            ridx = rows16 * DE
            ridx_p = [ridx + c if c else ridx for c in range(8)]
            dst = (g // 8) * 1024 + (g % 8) * 16
            for j in range(DE):
                vals = plsc.load_gather(
                    etab_v.at[pl.ds(8 * (j // 8), VE * DE - 8)],
                    [ridx_p[j % 8]])
                erows_v[bb, pl.ds(dst + (j // 8) * EB_T + (j % 8) * 128, 16)] = vals

    idx_start(0, 0)

    def pipe(it, carry):
        for b in range(2):
            i = 2 * it + b

            @pl.when(i + 1 < NCHE)
            def _():
                idx_start(i + 1, 1 - b)

            idx_wait(b)

            @pl.when(i >= 2)
            def _():
                write_wait(b)

            compute(b)
            write_start(i, b)
        return carry

    lax.fori_loop(0, NCHE // 2, pipe, 0)
    write_wait(0)
    write_wait(1)


def kernel(x, edge_attr, embed_x_W, embed_edge_W):
    mesh = plsc.VectorSubcoreMesh(core_axis_name="c", subcore_axis_name="s")
    f = pl.kernel(
        _body,
        out_type=[jax.ShapeDtypeStruct((NX, DX), jnp.float32),
                  jax.ShapeDtypeStruct((NE * DE,), jnp.float32)],
        mesh=mesh,
        compiler_params=pltpu.CompilerParams(
            use_tc_tiling_on_sc=False, needs_layout_passes=False),
        scratch_types=[
            pltpu.VMEM((NSUB, 128), jnp.int32),
            pltpu.VMEM((CX, DX), jnp.float32),
            pltpu.VMEM((VE * DE,), jnp.float32),
            pltpu.VMEM((2, CE), jnp.int32),
            pltpu.VMEM((2, 2 * EB_T), jnp.float32),
            pltpu.SemaphoreType.DMA,
            pltpu.SemaphoreType.DMA,
            pltpu.SemaphoreType.DMA,
            pltpu.SemaphoreType.DMA,
            pltpu.SemaphoreType.DMA,
        ],
    )
    out_x, out_e = f(x, edge_attr, embed_x_W, embed_edge_W.reshape(-1))
    out_e = out_e.reshape(2, NE // 128, 8, 128).transpose(1, 3, 0, 2)
    return (out_x, out_e.reshape(NE, DE))


# speedup vs baseline: 56.8078x; 2.0493x over previous
import jax
import jax.numpy as jnp
from jax import lax
from jax.experimental import pallas as pl
from jax.experimental.pallas import tpu as pltpu
from jax.experimental.pallas import tpu_sc as plsc

NC, NS = 2, 16
NW = NC * NS

NX, DX = 100000, 32
NE, DE = 3200000, 16
VE = 1000

CX = 1024
NSUB = CX // 128
NCHX = (NX + CX - 1) // CX
XLAST = NX - CX

NBLK = NE // 128
WBLK = 800
CE = 2048
NCHE = WBLK // 16
NG = CE // 16
EB_T = 16 * 8 * 128
ST = DE + 1


def _body(x_hbm, e_hbm, wx_hbm, we_hbm, outx_hbm, oute_hbm,
          xidx_v, xrows_v, etab_v, eidx_v, erows_v, sem, si0, si1, sw0, sw1):
    wid = lax.axis_index("s") * NC + lax.axis_index("c")

    for i in range(4):
        c = wid + NW * i

        @pl.when(c < NCHX)
        def _():
            base = pl.multiple_of(jnp.minimum(c * CX, XLAST), 8)
            for j in range(NSUB):
                pltpu.sync_copy(x_hbm.at[pl.ds(base + j * 128, 128)],
                                xidx_v.at[j])
            cps = [pltpu.async_copy(wx_hbm.at[xidx_v.at[j]],
                                    xrows_v.at[pl.ds(j * 128, 128)], sem)
                   for j in range(NSUB)]
            for cp in cps:
                cp.wait()
            pltpu.sync_copy(xrows_v, outx_hbm.at[pl.ds(base, CX)])

    pltpu.sync_copy(we_hbm, erows_v.at[0, pl.ds(0, VE * DE)])

    def trow(r, carry):
        etab_v[pl.ds(r * ST, DE)] = erows_v[0, pl.ds(r * DE, DE)]
        return carry

    lax.fori_loop(0, VE, trow, 0)
    wblk0 = jnp.minimum(wid * WBLK, NBLK - WBLK)
    si = (si0, si1)
    sw = (sw0, sw1)

    def idx_start(i, bb):
        b = pl.multiple_of((wblk0 + i * 16) * 128, 8)
        pltpu.async_copy(e_hbm.at[pl.ds(b, CE)], eidx_v.at[bb], si[bb])

    def idx_wait(bb):
        pltpu.make_async_copy(e_hbm.at[pl.ds(0, CE)], eidx_v.at[bb],
                              si[bb]).wait()

    def write_start(i, bb):
        blk = wblk0 + i * 16
        for t in range(2):
            pltpu.async_copy(
                erows_v.at[bb, pl.ds(t * EB_T, EB_T)],
                oute_hbm.at[pl.ds((t * NBLK + blk) * 1024, EB_T)], sw[bb])

    def write_wait(bb):
        pltpu.make_async_copy(oute_hbm.at[pl.ds(0, 2 * EB_T)],
                              erows_v.at[bb], sw[bb]).wait()

    def compute(bb):
        @plsc.parallel_loop(0, NG, unroll=4)
        def _(g):
            rows16 = eidx_v[bb, pl.ds(g * 16, 16)]
            ridx = rows16 * ST
            ridx_p = [ridx + c if c else ridx for c in range(8)]
            dst = (g // 8) * 1024 + (g % 8) * 16
            for j in range(DE):
                vals = plsc.load_gather(
                    etab_v.at[pl.ds(8 * (j // 8), VE * ST - 8)],
                    [ridx_p[j % 8]])
                erows_v[bb, pl.ds(dst + (j // 8) * EB_T + (j % 8) * 128, 16)] = vals

    idx_start(0, 0)

    def pipe(it, carry):
        for b in range(2):
            i = 2 * it + b

            @pl.when(i + 1 < NCHE)
            def _():
                idx_start(i + 1, 1 - b)

            idx_wait(b)

            @pl.when(i >= 2)
            def _():
                write_wait(b)

            compute(b)
            write_start(i, b)
        return carry

    lax.fori_loop(0, NCHE // 2, pipe, 0)
    write_wait(0)
    write_wait(1)


def kernel(x, edge_attr, embed_x_W, embed_edge_W):
    mesh = plsc.VectorSubcoreMesh(core_axis_name="c", subcore_axis_name="s")
    f = pl.kernel(
        _body,
        out_type=[jax.ShapeDtypeStruct((NX, DX), jnp.float32),
                  jax.ShapeDtypeStruct((NE * DE,), jnp.float32)],
        mesh=mesh,
        compiler_params=pltpu.CompilerParams(
            use_tc_tiling_on_sc=False, needs_layout_passes=False),
        scratch_types=[
            pltpu.VMEM((NSUB, 128), jnp.int32),
            pltpu.VMEM((CX, DX), jnp.float32),
            pltpu.VMEM((VE * ST,), jnp.float32),
            pltpu.VMEM((2, CE), jnp.int32),
            pltpu.VMEM((2, 2 * EB_T), jnp.float32),
            pltpu.SemaphoreType.DMA,
            pltpu.SemaphoreType.DMA,
            pltpu.SemaphoreType.DMA,
            pltpu.SemaphoreType.DMA,
            pltpu.SemaphoreType.DMA,
        ],
    )
    out_x, out_e = f(x, edge_attr, embed_x_W, embed_edge_W.reshape(-1))
    out_e = out_e.reshape(2, NE // 128, 8, 128).transpose(1, 3, 0, 2)
    return (out_x, out_e.reshape(NE, DE))
